# baseline (speedup 1.0000x reference)
import functools

import jax
import jax.numpy as jnp
from jax.experimental import pallas as pl
from jax.experimental.pallas import tpu as pltpu
from jax.experimental.pallas import tpu_sc as plsc

_L = 4
_B = 8
_T = 2048
_D = 256
_TTL = 1024
_N = _B * _T
_HALF_LIFE = 256.0
_BIG = (1 << 30)

_SCC = 2
_TCC = _B - _SCC
_NCORES = 2
_NSUB = 16
_NTILES = _NCORES * _NSUB
_TPL = _NTILES // _L
_RPT = _SCC * _T // _TPL
_CH = 128
_NCHUNK = _RPT // _CH


def _prep_kernel(scal_ref, p_ref, bs_ref, w_ref, wsn_ref, ls_ref):
    step = scal_ref[0]
    horizon = scal_ref[1]
    k = scal_ref[2]

    p = p_ref[...]
    bits = jax.lax.bitcast_convert_type(p, jnp.int32)
    s = bits ^ jax.lax.shift_right_arithmetic(bits, 31).astype(jnp.int32) & jnp.int32(0x7FFFFFFF)

    t = jnp.int32(-2147483648)
    for bit in range(30, -1, -1):
        tp = t + jnp.int32(1 << bit)
        cnt = jnp.sum((s >= tp).astype(jnp.int32))
        t = jnp.where(cnt >= k, tp, t)

    c_gt = jnp.sum((s > t).astype(jnp.int32))
    need = k - c_gt
    eq = s == t
    row = jax.lax.broadcasted_iota(jnp.int32, (_B, _T), 0)
    col = jax.lax.broadcasted_iota(jnp.int32, (_B, _T), 1)
    fi = row * _T + col
    sel_i = jnp.int32(0)
    for bit in range(13, -1, -1):
        ip = sel_i | jnp.int32(1 << bit)
        f = jnp.sum((eq & (fi <= ip)).astype(jnp.int32))
        sel_i = jnp.where(f <= need, ip, sel_i)

    mask = (s > t) | (eq & (fi <= sel_i))
    mf = mask.astype(jnp.float32)
    n_ev = jnp.sum(mf)
    inv_n = 1.0 / jnp.maximum(n_ev, 1.0)

    cntw = jnp.minimum(horizon, (_T - 1) - col)
    g = jnp.where(cntw > 0, mf / jnp.maximum(cntw, 1).astype(jnp.float32), 0.0)

    def shr(x, n):
        if n >= _T:
            return jnp.zeros_like(x)
        return jnp.concatenate([jnp.zeros((_B, n), dtype=x.dtype), x[:, :-n]], axis=1)

    big_g = g
    sh = 1
    while sh < _T:
        big_g = big_g + shr(big_g, sh)
        sh *= 2
    hp1 = jnp.minimum(horizon + 1, _T + 1)
    shifted = big_g
    for bit in range(12):
        amt = 1 << bit
        cond = ((hp1 >> bit) & 1) == 1
        shifted = jnp.where(cond, shr(shifted, amt), shifted)
    w_ref[...] = shr(big_g, 1) - shifted

    bsv = bs_ref[...]
    sidx = jax.lax.broadcasted_iota(jnp.int32, (_L, _TTL), 1)
    empty = bsv == jnp.int32(-1)
    first_empty = jnp.min(jnp.where(empty, sidx, _BIG), axis=1, keepdims=True)
    minval = jnp.min(bsv, axis=1, keepdims=True)
    first_min = jnp.min(jnp.where(bsv == minval, sidx, _BIG), axis=1, keepdims=True)
    slot = jnp.where(first_empty < _BIG, first_empty, first_min)

    age = jnp.maximum(step - bsv, 0).astype(jnp.float32)
    wgt = jnp.exp2(-age / _HALF_LIFE) * (bsv >= 0).astype(jnp.float32)
    wgt = jnp.where(sidx == slot, 0.0, wgt)
    wsum = jnp.sum(wgt, axis=1, keepdims=True) + 1.0
    wsn_ref[...] = wgt / wsum
    ls_ref[...] = (inv_n / wsum).reshape(_L, 1, 1)


def _reduce_kernel(w_ref, wsn_ref, ls_ref, x_ref, be_ref, out_ref):
    b = pl.program_id(1)
    x = x_ref[0, 0]
    wrow = w_ref[0]
    part = jax.lax.dot_general(
        wrow, x * x, (((1,), (0,)), ((), ())),
        preferred_element_type=jnp.float32)
    contrib = (ls_ref[0] * part)[None]

    @pl.when(b == 0)
    def _():
        be = be_ref[0]
        wsrow = wsn_ref[0]
        bank = jax.lax.dot_general(
            wsrow, be, (((1,), (0,)), ((), ())),
            preferred_element_type=jnp.float32)
        out_ref[...] = bank[None] + contrib

    @pl.when(b != 0)
    def _():
        out_ref[...] += contrib


def _sc_reduce_kernel(states_hbm, w_hbm, out_hbm,
                      xbuf0, xbuf1, wbuf, accbuf, sem0, sem1):
    c = jax.lax.axis_index("c")
    s = jax.lax.axis_index("s")
    wid = s * _NCORES + c
    layer = wid // _TPL
    j = wid % _TPL
    row0 = _TCC * _T + j * _RPT

    pltpu.sync_copy(w_hbm.at[pl.ds(row0, _RPT)], wbuf)

    bufs = [xbuf0, xbuf1]
    sems = [sem0, sem1]

    def start(g):
        return pltpu.async_copy(
            states_hbm.at[layer, pl.ds(row0 + g * _CH, _CH)],
            bufs[g % 2], sems[g % 2])

    pending = start(0)
    acc = tuple(jnp.zeros((16,), jnp.float32) for _ in range(_D // 16))
    for g in range(_NCHUNK):
        nxt = start(g + 1) if g + 1 < _NCHUNK else None
        pending.wait()
        xb = bufs[g % 2]
        base = g * _CH

        def group_body(gr, a):
            wv16 = wbuf[pl.ds(base + gr * 16, 16)]
            out = list(a)
            for rr in range(16):
                wbc = wv16.at[jnp.full((16,), rr, jnp.int32)].get(
                    mode='promise_in_bounds')
                for jj in range(_D // 16):
                    xv = xb[gr * 16 + rr, pl.ds(jj * 16, 16)]
                    out[jj] = out[jj] + wbc * xv * xv
            return tuple(out)

        acc = jax.lax.fori_loop(0, _CH // 16, group_body, acc)
        pending = nxt

    for jj in range(_D // 16):
        accbuf[pl.ds(jj * 16, 16)] = acc[jj]
    pltpu.sync_copy(accbuf, out_hbm.at[wid])


@jax.jit
def kernel(pressure, states, bank_evidence, bank_step, bank_event_count,
           baseline_future_energy, step, horizon_H, events_k):
    del bank_event_count, baseline_future_energy
    scal = jnp.stack([jnp.asarray(step, jnp.int32),
                      jnp.asarray(horizon_H, jnp.int32),
                      jnp.asarray(events_k, jnp.int32)])

    w, wsn, ls = pl.pallas_call(
        _prep_kernel,
        in_specs=[
            pl.BlockSpec(memory_space=pltpu.MemorySpace.SMEM),
            pl.BlockSpec(memory_space=pltpu.MemorySpace.VMEM),
            pl.BlockSpec(memory_space=pltpu.MemorySpace.VMEM),
        ],
        out_specs=[
            pl.BlockSpec(memory_space=pltpu.MemorySpace.VMEM),
            pl.BlockSpec(memory_space=pltpu.MemorySpace.VMEM),
            pl.BlockSpec(memory_space=pltpu.MemorySpace.VMEM),
        ],
        out_shape=[
            jax.ShapeDtypeStruct((_B, _T), jnp.float32),
            jax.ShapeDtypeStruct((_L, _TTL), jnp.float32),
            jax.ShapeDtypeStruct((_L, 1, 1), jnp.float32),
        ],
    )(scal, pressure, bank_step)

    w3 = w.reshape(_B, 1, _T)
    wsn3 = wsn.reshape(_L, 1, _TTL)

    sc_part = pl.kernel(
        _sc_reduce_kernel,
        out_type=jax.ShapeDtypeStruct((_NTILES, _D), jnp.float32),
        mesh=plsc.VectorSubcoreMesh(core_axis_name="c", subcore_axis_name="s"),
        scratch_types=[
            pltpu.VMEM((_CH, _D), jnp.float32),
            pltpu.VMEM((_CH, _D), jnp.float32),
            pltpu.VMEM((_RPT,), jnp.float32),
            pltpu.VMEM((_D,), jnp.float32),
            pltpu.SemaphoreType.DMA,
            pltpu.SemaphoreType.DMA,
        ],
    )(states.reshape(_L, _N, _D), w.reshape(_N))

    tc_score = pl.pallas_call(
        _reduce_kernel,
        grid=(_L, _TCC),
        in_specs=[
            pl.BlockSpec((1, 1, _T), lambda l, b: (b, 0, 0)),
            pl.BlockSpec((1, 1, _TTL), lambda l, b: (l, 0, 0)),
            pl.BlockSpec((1, 1, 1), lambda l, b: (l, 0, 0)),
            pl.BlockSpec((1, 1, _T, _D), lambda l, b: (l, b, 0, 0)),
            pl.BlockSpec((1, _TTL, _D), lambda l, b: (l, 0, 0)),
        ],
        out_specs=pl.BlockSpec((1, 1, _D), lambda l, b: (l, 0, 0)),
        out_shape=jax.ShapeDtypeStruct((_L, 1, _D), jnp.float32),
        compiler_params=pltpu.CompilerParams(
            dimension_semantics=("parallel", "arbitrary")),
    )(w3, wsn3, ls, states, bank_evidence)

    sc_sum = sc_part.reshape(_L, _TPL, _D).sum(axis=1)
    return tc_score.reshape(_L, _D) + ls.reshape(_L, 1) * sc_sum


# speedup vs baseline: 1.6622x; 1.6622x over previous
import functools

import jax
import jax.numpy as jnp
from jax.experimental import pallas as pl
from jax.experimental.pallas import tpu as pltpu

_L = 4
_B = 8
_T = 2048
_D = 256
_TTL = 1024
_N = _B * _T
_HALF_LIFE = 256.0
_BIG = (1 << 30)
_CPB = 2


def _prep(scal_ref, p_ref, bs_ref, w_scr, wsn_scr, ls_scr):
    step = scal_ref[0]
    horizon = scal_ref[1]
    k = scal_ref[2]

    p = p_ref[...]
    bits = jax.lax.bitcast_convert_type(p, jnp.int32)
    s = bits ^ jax.lax.shift_right_arithmetic(bits, 31).astype(jnp.int32) & jnp.int32(0x7FFFFFFF)

    t = jnp.int32(-2147483648)
    for bit in range(30, -1, -1):
        tp = t + jnp.int32(1 << bit)
        cnt = jnp.sum((s >= tp).astype(jnp.int32))
        t = jnp.where(cnt >= k, tp, t)

    c_gt = jnp.sum((s > t).astype(jnp.int32))
    need = k - c_gt
    eq = s == t
    row = jax.lax.broadcasted_iota(jnp.int32, (_B, _T), 0)
    col = jax.lax.broadcasted_iota(jnp.int32, (_B, _T), 1)
    fi = row * _T + col
    sel_i = jnp.int32(0)
    for bit in range(13, -1, -1):
        ip = sel_i | jnp.int32(1 << bit)
        f = jnp.sum((eq & (fi <= ip)).astype(jnp.int32))
        sel_i = jnp.where(f <= need, ip, sel_i)

    mask = (s > t) | (eq & (fi <= sel_i))
    mf = mask.astype(jnp.float32)
    n_ev = jnp.sum(mf)
    inv_n = 1.0 / jnp.maximum(n_ev, 1.0)

    cntw = jnp.minimum(horizon, (_T - 1) - col)
    g = jnp.where(cntw > 0, mf / jnp.maximum(cntw, 1).astype(jnp.float32), 0.0)

    def shr(x, n):
        if n >= _T:
            return jnp.zeros_like(x)
        return jnp.concatenate([jnp.zeros((_B, n), dtype=x.dtype), x[:, :-n]], axis=1)

    big_g = g
    sh = 1
    while sh < _T:
        big_g = big_g + shr(big_g, sh)
        sh *= 2
    hp1 = jnp.minimum(horizon + 1, _T + 1)
    shifted = big_g
    for bit in range(12):
        amt = 1 << bit
        cond = ((hp1 >> bit) & 1) == 1
        shifted = jnp.where(cond, shr(shifted, amt), shifted)
    w_scr[...] = shr(big_g, 1) - shifted

    bsv = bs_ref[...]
    sidx = jax.lax.broadcasted_iota(jnp.int32, (_L, _TTL), 1)
    empty = bsv == jnp.int32(-1)
    first_empty = jnp.min(jnp.where(empty, sidx, _BIG), axis=1, keepdims=True)
    minval = jnp.min(bsv, axis=1, keepdims=True)
    first_min = jnp.min(jnp.where(bsv == minval, sidx, _BIG), axis=1, keepdims=True)
    slot = jnp.where(first_empty < _BIG, first_empty, first_min)

    age = jnp.maximum(step - bsv, 0).astype(jnp.float32)
    wgt = jnp.exp2(-age / _HALF_LIFE) * (bsv >= 0).astype(jnp.float32)
    wgt = jnp.where(sidx == slot, 0.0, wgt)
    wsum = jnp.sum(wgt, axis=1, keepdims=True) + 1.0
    wsn_scr[...] = wgt / wsum
    ls_scr[...] = inv_n / wsum


def _fused_kernel(scal_ref, p_ref, bs_ref, x_ref, be_ref, out_ref,
                  w_scr, wsn_scr, ls_scr):
    l = pl.program_id(0)
    b = pl.program_id(1)

    @pl.when((l == 0) & (b == 0))
    def _():
        _prep(scal_ref, p_ref, bs_ref, w_scr, wsn_scr, ls_scr)

    part = jnp.zeros((1, _D), jnp.float32)
    for c in range(_CPB):
        x = x_ref[0, c]
        wrow = w_scr[pl.ds(b * _CPB + c, 1), :]
        part += jax.lax.dot_general(
            wrow, x * x, (((1,), (0,)), ((), ())),
            preferred_element_type=jnp.float32)
    contrib = (ls_scr[pl.ds(l, 1), :] * part)[None]

    @pl.when(b == 0)
    def _():
        be = be_ref[0]
        wsrow = wsn_scr[pl.ds(l, 1), :]
        bank = jax.lax.dot_general(
            wsrow, be, (((1,), (0,)), ((), ())),
            preferred_element_type=jnp.float32)
        out_ref[...] = bank[None] + contrib

    @pl.when(b != 0)
    def _():
        out_ref[...] += contrib


@jax.jit
def kernel(pressure, states, bank_evidence, bank_step, bank_event_count,
           baseline_future_energy, step, horizon_H, events_k):
    del bank_event_count, baseline_future_energy
    scal = jnp.stack([jnp.asarray(step, jnp.int32),
                      jnp.asarray(horizon_H, jnp.int32),
                      jnp.asarray(events_k, jnp.int32)])

    score = pl.pallas_call(
        _fused_kernel,
        grid=(_L, _B // _CPB),
        in_specs=[
            pl.BlockSpec(memory_space=pltpu.MemorySpace.SMEM),
            pl.BlockSpec((_B, _T), lambda l, b: (0, 0)),
            pl.BlockSpec((_L, _TTL), lambda l, b: (0, 0)),
            pl.BlockSpec((1, _CPB, _T, _D), lambda l, b: (l, b, 0, 0)),
            pl.BlockSpec((1, _TTL, _D), lambda l, b: (l, 0, 0)),
        ],
        out_specs=pl.BlockSpec((1, 1, _D), lambda l, b: (l, 0, 0)),
        out_shape=jax.ShapeDtypeStruct((_L, 1, _D), jnp.float32),
        scratch_shapes=[
            pltpu.VMEM((_B, _T), jnp.float32),
            pltpu.VMEM((_L, _TTL), jnp.float32),
            pltpu.VMEM((_L, 1), jnp.float32),
        ],
        compiler_params=pltpu.CompilerParams(
            dimension_semantics=("arbitrary", "arbitrary")),
    )(scal, pressure, bank_step, states, bank_evidence)

    return score.reshape(_L, _D)


# speedup vs baseline: 1.7335x; 1.0429x over previous
import functools

import jax
import jax.numpy as jnp
from jax.experimental import pallas as pl
from jax.experimental.pallas import tpu as pltpu

_L = 4
_B = 8
_T = 2048
_D = 256
_TTL = 1024
_N = _B * _T
_HALF_LIFE = 256.0
_BIG = (1 << 30)
_CPB = 4


def _prep(scal_ref, p_ref, bs_ref, w_scr, wsn_scr, ls_scr):
    step = scal_ref[0]
    horizon = scal_ref[1]
    k = scal_ref[2]

    p = p_ref[...]
    bits = jax.lax.bitcast_convert_type(p, jnp.int32)
    s = bits ^ jax.lax.shift_right_arithmetic(bits, 31).astype(jnp.int32) & jnp.int32(0x7FFFFFFF)

    t = jnp.int32(-2147483648)
    for bit in range(30, -1, -1):
        tp = t + jnp.int32(1 << bit)
        cnt = jnp.sum((s >= tp).astype(jnp.int32))
        t = jnp.where(cnt >= k, tp, t)

    c_gt = jnp.sum((s > t).astype(jnp.int32))
    need = k - c_gt
    eq = s == t
    row = jax.lax.broadcasted_iota(jnp.int32, (_B, _T), 0)
    col = jax.lax.broadcasted_iota(jnp.int32, (_B, _T), 1)
    fi = row * _T + col
    sel_i = jnp.int32(0)
    for bit in range(13, -1, -1):
        ip = sel_i | jnp.int32(1 << bit)
        f = jnp.sum((eq & (fi <= ip)).astype(jnp.int32))
        sel_i = jnp.where(f <= need, ip, sel_i)

    mask = (s > t) | (eq & (fi <= sel_i))
    mf = mask.astype(jnp.float32)
    n_ev = jnp.sum(mf)
    inv_n = 1.0 / jnp.maximum(n_ev, 1.0)

    cntw = jnp.minimum(horizon, (_T - 1) - col)
    g = jnp.where(cntw > 0, mf / jnp.maximum(cntw, 1).astype(jnp.float32), 0.0)

    def shr(x, n):
        if n >= _T:
            return jnp.zeros_like(x)
        return jnp.concatenate([jnp.zeros((_B, n), dtype=x.dtype), x[:, :-n]], axis=1)

    big_g = g
    sh = 1
    while sh < _T:
        big_g = big_g + shr(big_g, sh)
        sh *= 2
    hp1 = jnp.minimum(horizon + 1, _T + 1)
    shifted = big_g
    for bit in range(12):
        amt = 1 << bit
        cond = ((hp1 >> bit) & 1) == 1
        shifted = jnp.where(cond, shr(shifted, amt), shifted)
    w_scr[...] = shr(big_g, 1) - shifted

    bsv = bs_ref[...]
    sidx = jax.lax.broadcasted_iota(jnp.int32, (_L, _TTL), 1)
    empty = bsv == jnp.int32(-1)
    first_empty = jnp.min(jnp.where(empty, sidx, _BIG), axis=1, keepdims=True)
    minval = jnp.min(bsv, axis=1, keepdims=True)
    first_min = jnp.min(jnp.where(bsv == minval, sidx, _BIG), axis=1, keepdims=True)
    slot = jnp.where(first_empty < _BIG, first_empty, first_min)

    age = jnp.maximum(step - bsv, 0).astype(jnp.float32)
    wgt = jnp.exp2(-age / _HALF_LIFE) * (bsv >= 0).astype(jnp.float32)
    wgt = jnp.where(sidx == slot, 0.0, wgt)
    wsum = jnp.sum(wgt, axis=1, keepdims=True) + 1.0
    wsn_scr[...] = wgt / wsum
    ls_scr[...] = inv_n / wsum


def _fused_kernel(scal_ref, p_ref, bs_ref, x_ref, be_ref, out_ref,
                  w_scr, wsn_scr, ls_scr):
    l = pl.program_id(0)
    b = pl.program_id(1)

    @pl.when((l == 0) & (b == 0))
    def _():
        _prep(scal_ref, p_ref, bs_ref, w_scr, wsn_scr, ls_scr)

    part = jnp.zeros((1, _D), jnp.float32)
    for c in range(_CPB):
        x = x_ref[0, c]
        wrow = w_scr[pl.ds(b * _CPB + c, 1), :]
        part += jax.lax.dot_general(
            wrow, x * x, (((1,), (0,)), ((), ())),
            preferred_element_type=jnp.float32)
    contrib = (ls_scr[pl.ds(l, 1), :] * part)[None]

    @pl.when(b == 0)
    def _():
        be = be_ref[0]
        wsrow = wsn_scr[pl.ds(l, 1), :]
        bank = jax.lax.dot_general(
            wsrow, be, (((1,), (0,)), ((), ())),
            preferred_element_type=jnp.float32)
        out_ref[...] = bank[None] + contrib

    @pl.when(b != 0)
    def _():
        out_ref[...] += contrib


@jax.jit
def kernel(pressure, states, bank_evidence, bank_step, bank_event_count,
           baseline_future_energy, step, horizon_H, events_k):
    del bank_event_count, baseline_future_energy
    scal = jnp.stack([jnp.asarray(step, jnp.int32),
                      jnp.asarray(horizon_H, jnp.int32),
                      jnp.asarray(events_k, jnp.int32)])

    score = pl.pallas_call(
        _fused_kernel,
        grid=(_L, _B // _CPB),
        in_specs=[
            pl.BlockSpec(memory_space=pltpu.MemorySpace.SMEM),
            pl.BlockSpec((_B, _T), lambda l, b: (0, 0)),
            pl.BlockSpec((_L, _TTL), lambda l, b: (0, 0)),
            pl.BlockSpec((1, _CPB, _T, _D), lambda l, b: (l, b, 0, 0)),
            pl.BlockSpec((1, _TTL, _D), lambda l, b: (l, 0, 0)),
        ],
        out_specs=pl.BlockSpec((1, 1, _D), lambda l, b: (l, 0, 0)),
        out_shape=jax.ShapeDtypeStruct((_L, 1, _D), jnp.float32),
        scratch_shapes=[
            pltpu.VMEM((_B, _T), jnp.float32),
            pltpu.VMEM((_L, _TTL), jnp.float32),
            pltpu.VMEM((_L, 1), jnp.float32),
        ],
        compiler_params=pltpu.CompilerParams(
            dimension_semantics=("arbitrary", "arbitrary")),
    )(scal, pressure, bank_step, states, bank_evidence)

    return score.reshape(_L, _D)
